# baseline (speedup 1.0000x reference)
import jax
import jax.numpy as jnp
import numpy as np
from jax.experimental import pallas as pl
from jax.experimental.pallas import tpu as pltpu

_B = 262144
_DIM = 6
_HALF = 3
_K = 40
_H = 128
_NL = 8
_PI = float(np.pi)
_BR = 8192

_ATAN_C = (0.9999999987329571, -0.3333329490271314, 0.19998530422323615,
           -0.14264510232090435, 0.10954998354223984, -0.0841450751516909,
           0.05818360636033609, -0.03143228778537418, 0.011064244656339386,
           -0.0018295627827675104)


def _atan(t):
    a = jnp.abs(t)
    big = a > 1.0
    r = jnp.where(big, 1.0 / a, a)
    r2 = r * r
    p = jnp.full_like(r2, _ATAN_C[-1])
    for c in _ATAN_C[-2::-1]:
        p = p * r2 + c
    at = p * r
    at = jnp.where(big, (_PI / 2.0) - at, at)
    return jnp.where(t < 0.0, -at, at)


def _fused_body(x_ref, Aeff_ref, b1_ref, W2T_ref, b2_ref, W3T_ref, b3_ref,
                ck_ref, R4a_ref, R4b_ref, M8_ref, o_ref):
    f32 = jnp.float32
    xT = jnp.transpose(x_ref[...])
    xpT = jnp.clip(xT[:_HALF], -1.0, 1.0)
    vT = xT[_HALF:] * (1.0 / _PI)
    n2 = jnp.sum(vT * vT, axis=0, keepdims=True)
    den = jax.lax.rsqrt(jnp.maximum(1.0 - n2, 1e-12))
    ycT = _atan(vT * den) * (2.0 / _PI)
    ycT = jnp.where(n2 < 1.0, ycT, 0.0)
    ypT = jnp.concatenate([xpT, ycT], axis=0)

    for i in range(_NL):
        a = jnp.dot(Aeff_ref[i], ypT, preferred_element_type=f32)
        h = jnp.maximum(a[:_H] + b1_ref[i], 0.0).astype(jnp.bfloat16)
        h = jnp.maximum(
            jnp.dot(W2T_ref[i], h, preferred_element_type=f32) + b2_ref[i],
            0.0).astype(jnp.bfloat16)
        th = jnp.dot(W3T_ref[i], h, preferred_element_type=f32) + b3_ref[i]
        e = jnp.exp(jnp.clip(th, -60.0, 60.0))
        w = jnp.clip(a[_H:2 * _H] + ck_ref[...], 0.0, 1.0)
        f4 = (jnp.dot(R4a_ref[...], e * w, preferred_element_type=f32)
              + jnp.dot(R4b_ref[...], e, preferred_element_type=f32))
        yt = f4[:_HALF] / f4[_HALF:]
        ypT = jnp.concatenate([a[2 * _H:2 * _H + _HALF], yt], axis=0)
    o_ref[...] = jax.lax.dot_general(
        ypT, M8_ref[...], (((0,), (0,)), ((), ())),
        preferred_element_type=f32)


@jax.jit
def kernel(x, W1, b1, W2, b2, W3, b3, orders, perms):
    f32 = jnp.float32
    eye6 = jnp.eye(_DIM, dtype=f32)
    inv = jnp.argsort(orders, axis=1)
    perm_mats = [jnp.transpose(eye6[orders[0]])]
    for i in range(_NL - 1):
        c = inv[i][perms[i]][orders[i + 1]]
        perm_mats.append(jnp.transpose(eye6[c]))
    perm_mats.append(jnp.transpose(eye6[inv[_NL - 1]]))

    gidx = np.repeat(np.arange(_HALF), _K)
    klocal = np.tile(np.arange(_K), _HALF).astype(np.float32)
    E20 = np.zeros((_HALF, _H), dtype=np.float32)
    E20[gidx, np.arange(_HALF * _K)] = 20.0
    E20 = jnp.asarray(E20)
    Aeff_list = []
    for i in range(_NL):
        M = perm_mats[i]
        blk1 = jnp.transpose(M[:, :_HALF] @ W1[i])
        blk2 = jnp.transpose(M[:, _HALF:] @ E20)
        blk3 = jnp.transpose(M[:, :_HALF])
        Aeff_list.append(jnp.concatenate(
            [blk1, blk2, blk3, jnp.zeros((5, _DIM), f32)], axis=0))
    Aeff = jnp.stack(Aeff_list)

    b1c = b1[:, :, None]
    b2c = b2[:, :, None]
    b3c = jnp.pad(b3, ((0, 0), (0, _H - _HALF * _K)))[:, :, None]
    ck = np.full((_H, 1), -1.0, dtype=np.float32)
    ck[:_HALF * _K, 0] = 20.0 - klocal
    ck = jnp.asarray(ck)

    W2T = jnp.transpose(W2, (0, 2, 1)).astype(jnp.bfloat16)
    W3T = jnp.transpose(
        jnp.pad(W3, ((0, 0), (0, 0), (0, _H - _HALF * _K))),
        (0, 2, 1)).astype(jnp.bfloat16)

    R4a = np.zeros((_DIM, _H), dtype=np.float32)
    R4a[gidx, np.arange(_HALF * _K)] = 2.0
    R4b = np.zeros((_DIM, _H), dtype=np.float32)
    R4b[gidx, np.arange(_HALF * _K)] = -1.0
    R4b[_HALF + gidx, np.arange(_HALF * _K)] = 1.0
    R4a = jnp.asarray(R4a)
    R4b = jnp.asarray(R4b)

    M8 = perm_mats[_NL]

    grid = (_B // _BR,)
    out = pl.pallas_call(
        _fused_body,
        grid=grid,
        in_specs=[
            pl.BlockSpec((_BR, _DIM), lambda i: (i, 0)),
            pl.BlockSpec((_NL, 264, _DIM), lambda i: (0, 0, 0)),
            pl.BlockSpec((_NL, _H, 1), lambda i: (0, 0, 0)),
            pl.BlockSpec((_NL, _H, _H), lambda i: (0, 0, 0)),
            pl.BlockSpec((_NL, _H, 1), lambda i: (0, 0, 0)),
            pl.BlockSpec((_NL, _H, _H), lambda i: (0, 0, 0)),
            pl.BlockSpec((_NL, _H, 1), lambda i: (0, 0, 0)),
            pl.BlockSpec((_H, 1), lambda i: (0, 0)),
            pl.BlockSpec((_DIM, _H), lambda i: (0, 0)),
            pl.BlockSpec((_DIM, _H), lambda i: (0, 0)),
            pl.BlockSpec((_DIM, _DIM), lambda i: (0, 0)),
        ],
        out_specs=pl.BlockSpec((_BR, _DIM), lambda i: (i, 0)),
        out_shape=jax.ShapeDtypeStruct((_B, _DIM), f32),
        compiler_params=pltpu.CompilerParams(
            dimension_semantics=("parallel",)),
    )(x, Aeff, b1c, W2T, b2c, W3T, b3c, ck, R4a, R4b, M8)
    return out


# speedup vs baseline: 1.0592x; 1.0592x over previous
import jax
import jax.numpy as jnp
import numpy as np
from jax.experimental import pallas as pl
from jax.experimental.pallas import tpu as pltpu

_B = 262144
_DIM = 6
_HALF = 3
_K = 40
_H = 128
_NL = 8
_PI = float(np.pi)
_BR = 4096

_ATAN_C = (0.9999999987329571, -0.3333329490271314, 0.19998530422323615,
           -0.14264510232090435, 0.10954998354223984, -0.0841450751516909,
           0.05818360636033609, -0.03143228778537418, 0.011064244656339386,
           -0.0018295627827675104)


def _atan(t):
    a = jnp.abs(t)
    big = a > 1.0
    r = jnp.where(big, 1.0 / a, a)
    r2 = r * r
    p = jnp.full_like(r2, _ATAN_C[-1])
    for c in _ATAN_C[-2::-1]:
        p = p * r2 + c
    at = p * r
    at = jnp.where(big, (_PI / 2.0) - at, at)
    return jnp.where(t < 0.0, -at, at)


def _fused_body(x_ref, Aeff_ref, b1_ref, W2T_ref, b2_ref, W3T_ref, b3_ref,
                ck_ref, R4a_ref, R4b_ref, M8_ref, o_ref):
    f32 = jnp.float32
    xT = jnp.transpose(x_ref[...])
    xpT = jnp.clip(xT[:_HALF], -1.0, 1.0)
    vT = xT[_HALF:] * (1.0 / _PI)
    n2 = jnp.sum(vT * vT, axis=0, keepdims=True)
    den = jax.lax.rsqrt(jnp.maximum(1.0 - n2, 1e-12))
    ycT = _atan(vT * den) * (2.0 / _PI)
    ycT = jnp.where(n2 < 1.0, ycT, 0.0)
    ypT = jnp.concatenate([xpT, ycT], axis=0)

    bf16 = jnp.bfloat16
    for i in range(_NL):
        h1 = jnp.dot(Aeff_ref[i, :_H], ypT,
                     preferred_element_type=f32).astype(bf16)
        a = jnp.dot(Aeff_ref[i, _H:], ypT,
                    preferred_element_type=f32)
        h = jnp.maximum(h1 + b1_ref[i], 0.0)
        h = jnp.maximum(
            jnp.dot(W2T_ref[i], h,
                    preferred_element_type=f32).astype(bf16) + b2_ref[i],
            0.0)
        th = (jnp.dot(W3T_ref[i], h, preferred_element_type=f32).astype(bf16)
              + b3_ref[i])
        e = jnp.exp(jnp.clip(th, -60.0, 60.0))
        w = jnp.clip(a[:_H] + ck_ref[...], 0.0, 1.0).astype(bf16)
        f4 = (jnp.dot(R4a_ref[...], e * w, preferred_element_type=f32)
              + jnp.dot(R4b_ref[...], e, preferred_element_type=f32))
        yt = f4[:_HALF] / f4[_HALF:]
        ypT = jnp.concatenate([a[_H:_H + _HALF], yt], axis=0)
    o_ref[...] = jax.lax.dot_general(
        ypT, M8_ref[...], (((0,), (0,)), ((), ())),
        preferred_element_type=f32)


@jax.jit
def kernel(x, W1, b1, W2, b2, W3, b3, orders, perms):
    f32 = jnp.float32
    eye6 = jnp.eye(_DIM, dtype=f32)
    inv = jnp.argsort(orders, axis=1)
    perm_mats = [jnp.transpose(eye6[orders[0]])]
    for i in range(_NL - 1):
        c = inv[i][perms[i]][orders[i + 1]]
        perm_mats.append(jnp.transpose(eye6[c]))
    perm_mats.append(jnp.transpose(eye6[inv[_NL - 1]]))

    gidx = np.repeat(np.arange(_HALF), _K)
    klocal = np.tile(np.arange(_K), _HALF).astype(np.float32)
    E20 = np.zeros((_HALF, _H), dtype=np.float32)
    E20[gidx, np.arange(_HALF * _K)] = 20.0
    E20 = jnp.asarray(E20)
    Aeff_list = []
    for i in range(_NL):
        M = perm_mats[i]
        blk1 = jnp.transpose(M[:, :_HALF] @ W1[i])
        blk2 = jnp.transpose(M[:, _HALF:] @ E20)
        blk3 = jnp.transpose(M[:, :_HALF])
        Aeff_list.append(jnp.concatenate(
            [blk1, blk2, blk3, jnp.zeros((5, _DIM), f32)], axis=0))
    Aeff = jnp.stack(Aeff_list)

    b1c = b1[:, :, None].astype(jnp.bfloat16)
    b2c = b2[:, :, None].astype(jnp.bfloat16)
    b3c = jnp.pad(b3, ((0, 0), (0, _H - _HALF * _K)))[:, :, None].astype(
        jnp.bfloat16)
    ck = np.full((_H, 1), -1.0, dtype=np.float32)
    ck[:_HALF * _K, 0] = 20.0 - klocal
    ck = jnp.asarray(ck)

    W2T = jnp.transpose(W2, (0, 2, 1)).astype(jnp.bfloat16)
    W3T = jnp.transpose(
        jnp.pad(W3, ((0, 0), (0, 0), (0, _H - _HALF * _K))),
        (0, 2, 1)).astype(jnp.bfloat16)

    R4a = np.zeros((_DIM, _H), dtype=np.float32)
    R4a[gidx, np.arange(_HALF * _K)] = 2.0
    R4b = np.zeros((_DIM, _H), dtype=np.float32)
    R4b[gidx, np.arange(_HALF * _K)] = -1.0
    R4b[_HALF + gidx, np.arange(_HALF * _K)] = 1.0
    R4a = jnp.asarray(R4a).astype(jnp.bfloat16)
    R4b = jnp.asarray(R4b).astype(jnp.bfloat16)

    M8 = perm_mats[_NL]

    grid = (_B // _BR,)
    out = pl.pallas_call(
        _fused_body,
        grid=grid,
        in_specs=[
            pl.BlockSpec((_BR, _DIM), lambda i: (i, 0)),
            pl.BlockSpec((_NL, 264, _DIM), lambda i: (0, 0, 0)),
            pl.BlockSpec((_NL, _H, 1), lambda i: (0, 0, 0)),
            pl.BlockSpec((_NL, _H, _H), lambda i: (0, 0, 0)),
            pl.BlockSpec((_NL, _H, 1), lambda i: (0, 0, 0)),
            pl.BlockSpec((_NL, _H, _H), lambda i: (0, 0, 0)),
            pl.BlockSpec((_NL, _H, 1), lambda i: (0, 0, 0)),
            pl.BlockSpec((_H, 1), lambda i: (0, 0)),
            pl.BlockSpec((_DIM, _H), lambda i: (0, 0)),
            pl.BlockSpec((_DIM, _H), lambda i: (0, 0)),
            pl.BlockSpec((_DIM, _DIM), lambda i: (0, 0)),
        ],
        out_specs=pl.BlockSpec((_BR, _DIM), lambda i: (i, 0)),
        out_shape=jax.ShapeDtypeStruct((_B, _DIM), f32),
        compiler_params=pltpu.CompilerParams(
            dimension_semantics=("parallel",)),
    )(x, Aeff, b1c, W2T, b2c, W3T, b3c, ck, R4a, R4b, M8)
    return out


# speedup vs baseline: 1.0623x; 1.0029x over previous
import jax
import jax.numpy as jnp
import numpy as np
from jax.experimental import pallas as pl
from jax.experimental.pallas import tpu as pltpu

_B = 262144
_DIM = 6
_HALF = 3
_K = 40
_H = 128
_NL = 8
_PI = float(np.pi)
_BR = 4096

_ATAN_C = (0.9999999987329571, -0.3333329490271314, 0.19998530422323615,
           -0.14264510232090435, 0.10954998354223984, -0.0841450751516909,
           0.05818360636033609, -0.03143228778537418, 0.011064244656339386,
           -0.0018295627827675104)


def _atan(t):
    a = jnp.abs(t)
    big = a > 1.0
    r = jnp.where(big, 1.0 / a, a)
    r2 = r * r
    p = jnp.full_like(r2, _ATAN_C[-1])
    for c in _ATAN_C[-2::-1]:
        p = p * r2 + c
    at = p * r
    at = jnp.where(big, (_PI / 2.0) - at, at)
    return jnp.where(t < 0.0, -at, at)


def _fused_body(x_ref, A1_ref, Aeff_ref, b1_ref, W2T_ref, b2_ref, W3T_ref,
                b3_ref, ck_ref, R4a_ref, R4b_ref, M8_ref, o_ref):
    f32 = jnp.float32
    xT = jnp.transpose(x_ref[...])
    xpT = jnp.clip(xT[:_HALF], -1.0, 1.0)
    vT = xT[_HALF:] * (1.0 / _PI)
    n2 = jnp.sum(vT * vT, axis=0, keepdims=True)
    den = jax.lax.rsqrt(jnp.maximum(1.0 - n2, 1e-12))
    ycT = _atan(vT * den) * (2.0 / _PI)
    ycT = jnp.where(n2 < 1.0, ycT, 0.0)
    ypT = jnp.concatenate([xpT, ycT], axis=0)

    bf16 = jnp.bfloat16
    for i in range(_NL):
        yp16 = ypT.astype(bf16)
        h1 = jnp.dot(A1_ref[i], yp16,
                     preferred_element_type=f32).astype(bf16)
        a = jnp.dot(Aeff_ref[i], ypT,
                    preferred_element_type=f32)
        h = jnp.maximum(h1 + b1_ref[i], 0.0)
        h = jnp.maximum(
            jnp.dot(W2T_ref[i], h,
                    preferred_element_type=f32).astype(bf16) + b2_ref[i],
            0.0)
        th = (jnp.dot(W3T_ref[i], h, preferred_element_type=f32).astype(bf16)
              + b3_ref[i])
        e = jnp.exp(jnp.clip(th, -60.0, 60.0))
        w = jnp.clip(a[:_HALF * _K] + ck_ref[...], 0.0, 1.0).astype(bf16)
        f4 = (jnp.dot(R4a_ref[...], e[:_HALF * _K] * w,
                      preferred_element_type=f32)
              + jnp.dot(R4b_ref[...], e, preferred_element_type=f32))
        yt = f4[:_HALF] / f4[_HALF:]
        ypT = jnp.concatenate([a[_HALF * _K:_HALF * _K + _HALF], yt], axis=0)
    o_ref[...] = jax.lax.dot_general(
        ypT, M8_ref[...], (((0,), (0,)), ((), ())),
        preferred_element_type=f32)


@jax.jit
def kernel(x, W1, b1, W2, b2, W3, b3, orders, perms):
    f32 = jnp.float32
    eye6 = jnp.eye(_DIM, dtype=f32)
    inv = jnp.argsort(orders, axis=1)
    perm_mats = [jnp.transpose(eye6[orders[0]])]
    for i in range(_NL - 1):
        c = inv[i][perms[i]][orders[i + 1]]
        perm_mats.append(jnp.transpose(eye6[c]))
    perm_mats.append(jnp.transpose(eye6[inv[_NL - 1]]))

    gidx = np.repeat(np.arange(_HALF), _K)
    klocal = np.tile(np.arange(_K), _HALF).astype(np.float32)
    E20 = np.zeros((_HALF, _HALF * _K), dtype=np.float32)
    E20[gidx, np.arange(_HALF * _K)] = 20.0
    E20 = jnp.asarray(E20)
    A1_list, Aeff_list = [], []
    for i in range(_NL):
        M = perm_mats[i]
        A1_list.append(jnp.transpose(M[:, :_HALF] @ W1[i]))
        blk2 = jnp.transpose(M[:, _HALF:] @ E20)
        blk3 = jnp.transpose(M[:, :_HALF])
        Aeff_list.append(jnp.concatenate(
            [blk2, blk3, jnp.zeros((5, _DIM), f32)], axis=0))
    A1 = jnp.stack(A1_list).astype(jnp.bfloat16)
    Aeff = jnp.stack(Aeff_list)

    b1c = b1[:, :, None].astype(jnp.bfloat16)
    b2c = b2[:, :, None].astype(jnp.bfloat16)
    b3c = jnp.pad(b3, ((0, 0), (0, _H - _HALF * _K)))[:, :, None].astype(
        jnp.bfloat16)
    ck = jnp.asarray((20.0 - klocal)[:, None])

    W2T = jnp.transpose(W2, (0, 2, 1)).astype(jnp.bfloat16)
    W3T = jnp.transpose(
        jnp.pad(W3, ((0, 0), (0, 0), (0, _H - _HALF * _K))),
        (0, 2, 1)).astype(jnp.bfloat16)

    R4a = np.zeros((_DIM, _HALF * _K), dtype=np.float32)
    R4a[gidx, np.arange(_HALF * _K)] = 2.0
    R4b = np.zeros((_DIM, _H), dtype=np.float32)
    R4b[gidx, np.arange(_HALF * _K)] = -1.0
    R4b[_HALF + gidx, np.arange(_HALF * _K)] = 1.0
    R4a = jnp.asarray(R4a).astype(jnp.bfloat16)
    R4b = jnp.asarray(R4b).astype(jnp.bfloat16)

    M8 = perm_mats[_NL]

    grid = (_B // _BR,)
    out = pl.pallas_call(
        _fused_body,
        grid=grid,
        in_specs=[
            pl.BlockSpec((_BR, _DIM), lambda i: (i, 0)),
            pl.BlockSpec((_NL, _H, _DIM), lambda i: (0, 0, 0)),
            pl.BlockSpec((_NL, _H, _DIM), lambda i: (0, 0, 0)),
            pl.BlockSpec((_NL, _H, 1), lambda i: (0, 0, 0)),
            pl.BlockSpec((_NL, _H, _H), lambda i: (0, 0, 0)),
            pl.BlockSpec((_NL, _H, 1), lambda i: (0, 0, 0)),
            pl.BlockSpec((_NL, _H, _H), lambda i: (0, 0, 0)),
            pl.BlockSpec((_NL, _H, 1), lambda i: (0, 0, 0)),
            pl.BlockSpec((_HALF * _K, 1), lambda i: (0, 0)),
            pl.BlockSpec((_DIM, _HALF * _K), lambda i: (0, 0)),
            pl.BlockSpec((_DIM, _H), lambda i: (0, 0)),
            pl.BlockSpec((_DIM, _DIM), lambda i: (0, 0)),
        ],
        out_specs=pl.BlockSpec((_BR, _DIM), lambda i: (i, 0)),
        out_shape=jax.ShapeDtypeStruct((_B, _DIM), f32),
        compiler_params=pltpu.CompilerParams(
            dimension_semantics=("parallel",)),
    )(x, A1, Aeff, b1c, W2T, b2c, W3T, b3c, ck, R4a, R4b, M8)
    return out
